# baseline (speedup 1.0000x reference)
import functools

import jax
import jax.numpy as jnp
from jax import lax
from jax.experimental import pallas as pl
from jax.experimental.pallas import tpu as pltpu
from jax.experimental.pallas import tpu_sc as plsc

POOL = 7
PP = POOL * POOL
CH0 = 112
CH1 = 96
GPAD = CH0 + CH1
P0 = CH0 // 4
WROW = 128


def _prep(boxes, image_meta, feats):
    B, Nb, _ = boxes.shape
    C = feats[0].shape[-1]
    N = B * Nb

    hs = [f.shape[1] for f in feats]
    ws = [f.shape[2] for f in feats]
    sizes = [B * h * w for h, w in zip(hs, ws)]
    bases = [sum(sizes[:i]) for i in range(len(sizes))]

    table = jnp.concatenate([f.reshape(-1, C) for f in feats], axis=0)

    fb = boxes.reshape(-1, 4)
    bw = fb[:, 3] - fb[:, 1]
    bh = fb[:, 2] - fb[:, 0]
    image_shape = image_meta[0, 4:7]
    image_area = image_shape[0] * image_shape[1]
    eq = jnp.log(jnp.sqrt(jnp.maximum(bw * bh, 1e-12)) * jnp.sqrt(image_area) / 224.0) / jnp.log(2.0)
    levels = jnp.maximum(2, jnp.minimum(4 + jnp.round(eq).astype(jnp.int32), 5))
    li = levels - 2

    h_i = jnp.take(jnp.asarray(hs, jnp.int32), li)
    w_i = jnp.take(jnp.asarray(ws, jnp.int32), li)
    hw_i = jnp.take(jnp.asarray([h * w for h, w in zip(hs, ws)], jnp.int32), li)
    base = jnp.take(jnp.asarray(bases, jnp.int32), li)
    bb = jnp.repeat(jnp.arange(B, dtype=jnp.int32), Nb)
    base = base + bb * hw_i

    hf = h_i.astype(jnp.float32)
    wf = w_i.astype(jnp.float32)
    g = jnp.arange(POOL, dtype=jnp.float32)
    ys = fb[:, 0][:, None] * (hf - 1.0)[:, None] + g[None, :] * ((fb[:, 2] - fb[:, 0]) * (hf - 1.0) / (POOL - 1))[:, None]
    xs = fb[:, 1][:, None] * (wf - 1.0)[:, None] + g[None, :] * ((fb[:, 3] - fb[:, 1]) * (wf - 1.0) / (POOL - 1))[:, None]
    y0f = jnp.floor(ys)
    x0f = jnp.floor(xs)
    y0 = jnp.clip(y0f.astype(jnp.int32), 0, h_i[:, None] - 1)
    y1 = jnp.clip(y0 + 1, 0, h_i[:, None] - 1)
    x0 = jnp.clip(x0f.astype(jnp.int32), 0, w_i[:, None] - 1)
    x1 = jnp.clip(x0 + 1, 0, w_i[:, None] - 1)
    wy = ys - y0f
    wx = xs - x0f

    ry0 = base[:, None] + y0 * w_i[:, None]
    ry1 = base[:, None] + y1 * w_i[:, None]
    tl = ry0[:, :, None] + x0[:, None, :]
    tr = ry0[:, :, None] + x1[:, None, :]
    bl = ry1[:, :, None] + x0[:, None, :]
    br = ry1[:, :, None] + x1[:, None, :]
    inter = jnp.stack([tl, tr, bl, br], axis=-1).reshape(N, 4 * PP).astype(jnp.int32)
    pad = jnp.broadcast_to(inter[:, -1:], (N, GPAD - 4 * PP))
    idx = jnp.concatenate([inter, pad], axis=1)

    wx_p = jnp.broadcast_to(wx[:, None, :], (N, POOL, POOL)).reshape(N, PP)
    wy_p = jnp.broadcast_to(wy[:, :, None], (N, POOL, POOL)).reshape(N, PP)
    zeros15 = jnp.zeros((N, 64 - PP), jnp.float32)
    wrow = jnp.concatenate([wx_p, zeros15, wy_p, zeros15], axis=1)
    return table, idx, wrow


def kernel(boxes, image_meta, feat_p2, feat_p3, feat_p4, feat_p5):
    feats = [feat_p2, feat_p3, feat_p4, feat_p5]
    B, Nb, _ = boxes.shape
    C = feats[0].shape[-1]
    N = B * Nb
    assert C % 16 == 0

    table, idx, wrow = _prep(boxes, image_meta, feats)

    info = plsc.get_sparse_core_info()
    NC, NS = info.num_cores, info.num_subcores
    NW = NC * NS
    per_worker = -(-N // NW)
    NPAD = NW * per_worker
    npad = NPAD - N

    idx_t = jnp.concatenate([idx, jnp.zeros((npad, GPAD), jnp.int32)], axis=0)
    idx_t = idx_t.reshape(NW, per_worker * GPAD)
    wrow_t = jnp.concatenate([wrow, jnp.zeros((npad, WROW), jnp.float32)], axis=0)
    wrow_t = wrow_t.reshape(NW, per_worker * WROW)

    mesh = plsc.VectorSubcoreMesh(core_axis_name="c", subcore_axis_name="s")

    @functools.partial(
        pl.kernel,
        mesh=mesh,
        out_type=jax.ShapeDtypeStruct((NPAD, PP, C), jnp.float32),
        scratch_types=[
            pltpu.VMEM((per_worker * GPAD,), jnp.int32),
            pltpu.VMEM((per_worker * WROW,), jnp.float32),
            pltpu.VMEM((CH0, C), jnp.float32),
            pltpu.VMEM((CH1, C), jnp.float32),
            pltpu.VMEM((PP, C), jnp.float32),
            pltpu.VMEM((PP, C), jnp.float32),
            pltpu.SemaphoreType.DMA,
            pltpu.SemaphoreType.DMA,
            pltpu.SemaphoreType.DMA,
            pltpu.SemaphoreType.DMA,
        ],
    )
    def sc_pool(table_h, idx_h, w_h, out_h, idx_v, w_v, buf0, buf1,
                out0, out1, sem0, sem1, osem0, osem1):
        wid = lax.axis_index("s") * NC + lax.axis_index("c")
        base_box = wid * per_worker

        pltpu.sync_copy(idx_h.at[wid], idx_v)
        pltpu.sync_copy(w_h.at[wid], w_v)

        def gather0(j):
            pltpu.async_copy(
                table_h.at[idx_v.at[pl.ds(j * GPAD, CH0)]], buf0, sem0)

        def gather1(j):
            pltpu.async_copy(
                table_h.at[idx_v.at[pl.ds(j * GPAD + CH0, CH1)]], buf1, sem1)

        gather0(0)

        dnums = lax.GatherDimensionNumbers(
            offset_dims=(), collapsed_slice_dims=(0,), start_index_map=(0,))

        def bcast_lane(vec16, lane):
            idxv = jnp.broadcast_to(lane, (16,)).astype(jnp.int32)
            return lax.gather(vec16, idxv[:, None], dnums, (1,),
                              mode=lax.GatherScatterMode.PROMISE_IN_BOUNDS)

        def combine(p, src, r, outb, wb):
            chunk = (p // 16) * 16
            lane = p - chunk
            wxc = w_v[pl.ds(wb + chunk, 16)]
            wyc = w_v[pl.ds(wb + 64 + chunk, 16)]
            wxp = bcast_lane(wxc, lane)
            wyp = bcast_lane(wyc, lane)
            for ch in range(C // 16):
                s = pl.ds(ch * 16, 16)
                tl = src[r, s]
                tr = src[r + 1, s]
                bl = src[r + 2, s]
                br = src[r + 3, s]
                top = tl + (tr - tl) * wxp
                bot = bl + (br - bl) * wxp
                outb[p, s] = top + (bot - top) * wyp

        def box_impl(j, outb, osem):
            wb = j * WROW

            pltpu.make_async_copy(table_h.at[pl.ds(0, CH0)], buf0, sem0).wait()
            gather1(j)

            def pt0(p, c):
                combine(p, buf0, p * 4, outb, wb)
                return c
            lax.fori_loop(0, P0, pt0, 0)

            pltpu.make_async_copy(table_h.at[pl.ds(0, CH1)], buf1, sem1).wait()

            @pl.when(j + 1 < per_worker)
            def _():
                gather0(j + 1)

            def pt1(p, c):
                combine(p, buf1, p * 4 - CH0, outb, wb)
                return c
            lax.fori_loop(P0, PP, pt1, 0)

            @pl.when(j >= 2)
            def _():
                pltpu.make_async_copy(out_h.at[0], outb, osem).wait()
            pltpu.async_copy(outb, out_h.at[base_box + j], osem)

        def box_body(j, carry):
            @pl.when(j % 2 == 0)
            def _():
                box_impl(j, out0, osem0)

            @pl.when(j % 2 == 1)
            def _():
                box_impl(j, out1, osem1)

            return carry

        lax.fori_loop(0, per_worker, box_body, 0)

        if per_worker >= 1:
            pltpu.make_async_copy(out_h.at[0], out0, osem0).wait()
        if per_worker >= 2:
            pltpu.make_async_copy(out_h.at[0], out1, osem1).wait()

    out = sc_pool(table, idx_t, wrow_t)
    return out[:N].reshape(B, Nb, POOL, POOL, C)


# speedup vs baseline: 1.0025x; 1.0025x over previous
import functools

import jax
import jax.numpy as jnp
from jax import lax
from jax.experimental import pallas as pl
from jax.experimental.pallas import tpu as pltpu
from jax.experimental.pallas import tpu_sc as plsc

POOL = 7
PP = POOL * POOL
CH0 = 112
CH1 = 96
GPAD = CH0 + CH1
P0 = CH0 // 4
WROW = 128


def _prep(boxes, image_meta, feats):
    B, Nb, _ = boxes.shape
    C = feats[0].shape[-1]
    N = B * Nb

    hs = [f.shape[1] for f in feats]
    ws = [f.shape[2] for f in feats]
    sizes = [B * h * w for h, w in zip(hs, ws)]
    bases = [sum(sizes[:i]) for i in range(len(sizes))]

    table = jnp.concatenate([f.reshape(-1, C) for f in feats], axis=0)

    fb = boxes.reshape(-1, 4)
    bw = fb[:, 3] - fb[:, 1]
    bh = fb[:, 2] - fb[:, 0]
    image_shape = image_meta[0, 4:7]
    image_area = image_shape[0] * image_shape[1]
    eq = jnp.log(jnp.sqrt(jnp.maximum(bw * bh, 1e-12)) * jnp.sqrt(image_area) / 224.0) / jnp.log(2.0)
    levels = jnp.maximum(2, jnp.minimum(4 + jnp.round(eq).astype(jnp.int32), 5))
    li = levels - 2

    h_i = jnp.take(jnp.asarray(hs, jnp.int32), li)
    w_i = jnp.take(jnp.asarray(ws, jnp.int32), li)
    hw_i = jnp.take(jnp.asarray([h * w for h, w in zip(hs, ws)], jnp.int32), li)
    base = jnp.take(jnp.asarray(bases, jnp.int32), li)
    bb = jnp.repeat(jnp.arange(B, dtype=jnp.int32), Nb)
    base = base + bb * hw_i

    hf = h_i.astype(jnp.float32)
    wf = w_i.astype(jnp.float32)
    g = jnp.arange(POOL, dtype=jnp.float32)
    ys = fb[:, 0][:, None] * (hf - 1.0)[:, None] + g[None, :] * ((fb[:, 2] - fb[:, 0]) * (hf - 1.0) / (POOL - 1))[:, None]
    xs = fb[:, 1][:, None] * (wf - 1.0)[:, None] + g[None, :] * ((fb[:, 3] - fb[:, 1]) * (wf - 1.0) / (POOL - 1))[:, None]
    y0f = jnp.floor(ys)
    x0f = jnp.floor(xs)
    y0 = jnp.clip(y0f.astype(jnp.int32), 0, h_i[:, None] - 1)
    y1 = jnp.clip(y0 + 1, 0, h_i[:, None] - 1)
    x0 = jnp.clip(x0f.astype(jnp.int32), 0, w_i[:, None] - 1)
    x1 = jnp.clip(x0 + 1, 0, w_i[:, None] - 1)
    wy = ys - y0f
    wx = xs - x0f

    ry0 = base[:, None] + y0 * w_i[:, None]
    ry1 = base[:, None] + y1 * w_i[:, None]
    tl = ry0[:, :, None] + x0[:, None, :]
    tr = ry0[:, :, None] + x1[:, None, :]
    bl = ry1[:, :, None] + x0[:, None, :]
    br = ry1[:, :, None] + x1[:, None, :]
    inter = jnp.stack([tl, tr, bl, br], axis=-1).reshape(N, 4 * PP).astype(jnp.int32)
    pad = jnp.broadcast_to(inter[:, -1:], (N, GPAD - 4 * PP))
    idx = jnp.concatenate([inter, pad], axis=1)

    wx_p = jnp.broadcast_to(wx[:, None, :], (N, POOL, POOL)).reshape(N, PP)
    wy_p = jnp.broadcast_to(wy[:, :, None], (N, POOL, POOL)).reshape(N, PP)
    zeros15 = jnp.zeros((N, 64 - PP), jnp.float32)
    wrow = jnp.concatenate([wx_p, zeros15, wy_p, zeros15], axis=1)
    return table, idx, wrow


def kernel(boxes, image_meta, feat_p2, feat_p3, feat_p4, feat_p5):
    feats = [feat_p2, feat_p3, feat_p4, feat_p5]
    B, Nb, _ = boxes.shape
    C = feats[0].shape[-1]
    N = B * Nb
    assert C % 16 == 0

    table, idx, wrow = _prep(boxes, image_meta, feats)

    info = plsc.get_sparse_core_info()
    NC, NS = info.num_cores, info.num_subcores
    NW = NC * NS
    per_worker = -(-N // NW)
    NPAD = NW * per_worker
    npad = NPAD - N

    idx_t = jnp.concatenate([idx, jnp.zeros((npad, GPAD), jnp.int32)], axis=0)
    idx_t = idx_t.reshape(NW, per_worker * GPAD)
    wrow_t = jnp.concatenate([wrow, jnp.zeros((npad, WROW), jnp.float32)], axis=0)
    wrow_t = wrow_t.reshape(NW, per_worker * WROW)

    mesh = plsc.VectorSubcoreMesh(core_axis_name="c", subcore_axis_name="s")

    @functools.partial(
        pl.kernel,
        mesh=mesh,
        out_type=jax.ShapeDtypeStruct((NPAD, PP, C), jnp.float32),
        scratch_types=[
            pltpu.VMEM((per_worker * GPAD,), jnp.int32),
            pltpu.VMEM((per_worker * WROW,), jnp.float32),
            pltpu.VMEM((CH0, C), jnp.float32),
            pltpu.VMEM((CH1, C), jnp.float32),
            pltpu.VMEM((PP, C), jnp.float32),
            pltpu.VMEM((PP, C), jnp.float32),
            pltpu.SemaphoreType.DMA,
            pltpu.SemaphoreType.DMA,
            pltpu.SemaphoreType.DMA,
            pltpu.SemaphoreType.DMA,
        ],
    )
    def sc_pool(table_h, idx_h, w_h, out_h, idx_v, w_v, buf0, buf1,
                out0, out1, sem0, sem1, osem0, osem1):
        wid = lax.axis_index("s") * NC + lax.axis_index("c")
        base_box = wid * per_worker

        pltpu.sync_copy(idx_h.at[wid], idx_v)
        pltpu.sync_copy(w_h.at[wid], w_v)

        def gather0(j):
            pltpu.async_copy(
                table_h.at[idx_v.at[pl.ds(j * GPAD, CH0)]], buf0, sem0)

        def gather1(j):
            pltpu.async_copy(
                table_h.at[idx_v.at[pl.ds(j * GPAD + CH0, CH1)]], buf1, sem1)

        gather0(0)

        dnums = lax.GatherDimensionNumbers(
            offset_dims=(), collapsed_slice_dims=(0,), start_index_map=(0,))

        def bcast_lane(vec16, lane):
            idxv = jnp.broadcast_to(lane, (16,)).astype(jnp.int32)
            return lax.gather(vec16, idxv[:, None], dnums, (1,),
                              mode=lax.GatherScatterMode.PROMISE_IN_BOUNDS)

        def combine(p, src, r, outb, wb):
            chunk = (p // 16) * 16
            lane = p - chunk
            wxc = w_v[pl.ds(wb + chunk, 16)]
            wyc = w_v[pl.ds(wb + 64 + chunk, 16)]
            wxp = bcast_lane(wxc, lane)
            wyp = bcast_lane(wyc, lane)
            for ch in range(C // 16):
                s = pl.ds(ch * 16, 16)
                tl = src[r, s]
                tr = src[r + 1, s]
                bl = src[r + 2, s]
                br = src[r + 3, s]
                top = tl + (tr - tl) * wxp
                bot = bl + (br - bl) * wxp
                outb[p, s] = top + (bot - top) * wyp

        def box_impl(j, outb, osem):
            wb = j * WROW

            gather1(j)
            pltpu.make_async_copy(table_h.at[pl.ds(0, CH0)], buf0, sem0).wait()

            def pt0(p, c):
                combine(p, buf0, p * 4, outb, wb)
                return c
            lax.fori_loop(0, P0, pt0, 0)

            pltpu.make_async_copy(table_h.at[pl.ds(0, CH1)], buf1, sem1).wait()

            @pl.when(j + 1 < per_worker)
            def _():
                gather0(j + 1)

            def pt1(p, c):
                combine(p, buf1, p * 4 - CH0, outb, wb)
                return c
            lax.fori_loop(P0, PP, pt1, 0)

            @pl.when(j >= 2)
            def _():
                pltpu.make_async_copy(out_h.at[0], outb, osem).wait()
            pltpu.async_copy(outb, out_h.at[base_box + j], osem)

        def box_body(j, carry):
            @pl.when(j % 2 == 0)
            def _():
                box_impl(j, out0, osem0)

            @pl.when(j % 2 == 1)
            def _():
                box_impl(j, out1, osem1)

            return carry

        lax.fori_loop(0, per_worker, box_body, 0)

        if per_worker >= 1:
            pltpu.make_async_copy(out_h.at[0], out0, osem0).wait()
        if per_worker >= 2:
            pltpu.make_async_copy(out_h.at[0], out1, osem1).wait()

    out = sc_pool(table, idx_t, wrow_t)
    return out[:N].reshape(B, Nb, POOL, POOL, C)


# speedup vs baseline: 1.6493x; 1.6453x over previous
import functools

import jax
import jax.numpy as jnp
from jax import lax
from jax.experimental import pallas as pl
from jax.experimental.pallas import tpu as pltpu
from jax.experimental.pallas import tpu_sc as plsc

POOL = 7
PP = POOL * POOL
CH0 = 112
CH1 = 96
GPAD = CH0 + CH1
P0 = CH0 // 4
WROW = 128


def _prep(boxes, image_meta, feats):
    B, Nb, _ = boxes.shape
    C = feats[0].shape[-1]
    N = B * Nb

    hs = [f.shape[1] for f in feats]
    ws = [f.shape[2] for f in feats]

    fb = boxes.reshape(-1, 4)
    bw = fb[:, 3] - fb[:, 1]
    bh = fb[:, 2] - fb[:, 0]
    image_shape = image_meta[0, 4:7]
    image_area = image_shape[0] * image_shape[1]
    eq = jnp.log(jnp.sqrt(jnp.maximum(bw * bh, 1e-12)) * jnp.sqrt(image_area) / 224.0) / jnp.log(2.0)
    levels = jnp.maximum(2, jnp.minimum(4 + jnp.round(eq).astype(jnp.int32), 5))
    li = levels - 2

    h_i = jnp.take(jnp.asarray(hs, jnp.int32), li)
    w_i = jnp.take(jnp.asarray(ws, jnp.int32), li)
    hw_i = jnp.take(jnp.asarray([h * w for h, w in zip(hs, ws)], jnp.int32), li)
    bb = jnp.repeat(jnp.arange(B, dtype=jnp.int32), Nb)
    base = bb * hw_i

    hf = h_i.astype(jnp.float32)
    wf = w_i.astype(jnp.float32)
    g = jnp.arange(POOL, dtype=jnp.float32)
    ys = fb[:, 0][:, None] * (hf - 1.0)[:, None] + g[None, :] * ((fb[:, 2] - fb[:, 0]) * (hf - 1.0) / (POOL - 1))[:, None]
    xs = fb[:, 1][:, None] * (wf - 1.0)[:, None] + g[None, :] * ((fb[:, 3] - fb[:, 1]) * (wf - 1.0) / (POOL - 1))[:, None]
    y0f = jnp.floor(ys)
    x0f = jnp.floor(xs)
    y0 = jnp.clip(y0f.astype(jnp.int32), 0, h_i[:, None] - 1)
    y1 = jnp.clip(y0 + 1, 0, h_i[:, None] - 1)
    x0 = jnp.clip(x0f.astype(jnp.int32), 0, w_i[:, None] - 1)
    x1 = jnp.clip(x0 + 1, 0, w_i[:, None] - 1)
    wy = ys - y0f
    wx = xs - x0f

    ry0 = base[:, None] + y0 * w_i[:, None]
    ry1 = base[:, None] + y1 * w_i[:, None]
    tl = ry0[:, :, None] + x0[:, None, :]
    tr = ry0[:, :, None] + x1[:, None, :]
    bl = ry1[:, :, None] + x0[:, None, :]
    br = ry1[:, :, None] + x1[:, None, :]
    inter = jnp.stack([tl, tr, bl, br], axis=-1).reshape(N, 4 * PP).astype(jnp.int32)
    pad = jnp.broadcast_to(inter[:, -1:], (N, GPAD - 4 * PP))
    idx = jnp.concatenate([inter, pad], axis=1)

    wx_p = jnp.broadcast_to(wx[:, None, :], (N, POOL, POOL)).reshape(N, PP)
    wy_p = jnp.broadcast_to(wy[:, :, None], (N, POOL, POOL)).reshape(N, PP)
    zeros15 = jnp.zeros((N, 64 - PP), jnp.float32)
    zeros14 = jnp.zeros((N, 63 - PP), jnp.float32)
    lvl_f = li.astype(jnp.float32)[:, None]
    wrow = jnp.concatenate([wx_p, zeros14, lvl_f, wy_p, zeros15], axis=1)
    return idx, wrow


def kernel(boxes, image_meta, feat_p2, feat_p3, feat_p4, feat_p5):
    feats = [feat_p2, feat_p3, feat_p4, feat_p5]
    B, Nb, _ = boxes.shape
    C = feats[0].shape[-1]
    N = B * Nb
    assert C % 16 == 0

    idx, wrow = _prep(boxes, image_meta, feats)

    info = plsc.get_sparse_core_info()
    NC, NS = info.num_cores, info.num_subcores
    NW = NC * NS
    per_worker = -(-N // NW)
    NPAD = NW * per_worker
    npad = NPAD - N

    idx_t = jnp.concatenate([idx, jnp.zeros((npad, GPAD), jnp.int32)], axis=0)
    idx_t = idx_t.reshape(NW, per_worker * GPAD)
    wrow_t = jnp.concatenate([wrow, jnp.zeros((npad, WROW), jnp.float32)], axis=0)
    wrow_t = wrow_t.reshape(NW, per_worker * WROW)

    flats = [f.reshape(-1, C) for f in feats]

    mesh = plsc.VectorSubcoreMesh(core_axis_name="c", subcore_axis_name="s")

    @functools.partial(
        pl.kernel,
        mesh=mesh,
        out_type=jax.ShapeDtypeStruct((N, PP, C), jnp.float32),
        scratch_types=[
            pltpu.VMEM((per_worker * GPAD,), jnp.int32),
            pltpu.VMEM((per_worker * WROW,), jnp.float32),
            pltpu.VMEM((CH0, C), jnp.float32),
            pltpu.VMEM((CH1, C), jnp.float32),
            pltpu.VMEM((PP, C), jnp.float32),
            pltpu.VMEM((PP, C), jnp.float32),
            pltpu.SemaphoreType.DMA,
            pltpu.SemaphoreType.DMA,
            pltpu.SemaphoreType.DMA,
            pltpu.SemaphoreType.DMA,
        ],
    )
    def sc_pool(fp2_h, fp3_h, fp4_h, fp5_h, idx_h, w_h, out_h,
                idx_v, w_v, buf0, buf1,
                out0, out1, sem0, sem1, osem0, osem1):
        wid = lax.axis_index("s") * NC + lax.axis_index("c")
        base_box = wid * per_worker
        fhs = [fp2_h, fp3_h, fp4_h, fp5_h]

        pltpu.sync_copy(idx_h.at[wid], idx_v)
        pltpu.sync_copy(w_h.at[wid], w_v)

        dnums = lax.GatherDimensionNumbers(
            offset_dims=(), collapsed_slice_dims=(0,), start_index_map=(0,))

        def bcast_lane(vec16, lane):
            idxv = jnp.broadcast_to(lane, (16,)).astype(jnp.int32)
            return lax.gather(vec16, idxv[:, None], dnums, (1,),
                              mode=lax.GatherScatterMode.PROMISE_IN_BOUNDS)

        def box_level(j):
            lv = w_v[pl.ds(j * WROW + 48, 16)]
            return lv[15]

        def gather0(j):
            lvl = box_level(j)
            for l, fh in enumerate(fhs):
                @pl.when(lvl == jnp.float32(l))
                def _():
                    pltpu.async_copy(
                        fh.at[idx_v.at[pl.ds(j * GPAD, CH0)]], buf0, sem0)

        def gather1(j):
            lvl = box_level(j)
            for l, fh in enumerate(fhs):
                @pl.when(lvl == jnp.float32(l))
                def _():
                    pltpu.async_copy(
                        fh.at[idx_v.at[pl.ds(j * GPAD + CH0, CH1)]],
                        buf1, sem1)

        @pl.when(base_box < N)
        def _():
            gather0(0)

        def combine(p, src, r, outb, wb):
            chunk = (p // 16) * 16
            lane = p - chunk
            wxc = w_v[pl.ds(wb + chunk, 16)]
            wyc = w_v[pl.ds(wb + 64 + chunk, 16)]
            wxp = bcast_lane(wxc, lane)
            wyp = bcast_lane(wyc, lane)
            for ch in range(C // 16):
                s = pl.ds(ch * 16, 16)
                tl = src[r, s]
                tr = src[r + 1, s]
                bl = src[r + 2, s]
                br = src[r + 3, s]
                top = tl + (tr - tl) * wxp
                bot = bl + (br - bl) * wxp
                outb[p, s] = top + (bot - top) * wyp

        def box_impl(j, outb, osem):
            wb = j * WROW

            gather1(j)
            pltpu.make_async_copy(fp2_h.at[pl.ds(0, CH0)], buf0, sem0).wait()

            def pt0(p, c):
                combine(p, buf0, p * 4, outb, wb)
                return c
            lax.fori_loop(0, P0, pt0, 0)

            pltpu.make_async_copy(fp2_h.at[pl.ds(0, CH1)], buf1, sem1).wait()

            @pl.when((j + 1 < per_worker) & (base_box + j + 1 < N))
            def _():
                gather0(j + 1)

            def pt1(p, c):
                combine(p, buf1, p * 4 - CH0, outb, wb)
                return c
            lax.fori_loop(P0, PP, pt1, 0)

            @pl.when(j >= 2)
            def _():
                pltpu.make_async_copy(out_h.at[0], outb, osem).wait()
            pltpu.async_copy(outb, out_h.at[base_box + j], osem)

        def box_body(j, carry):
            @pl.when(base_box + j < N)
            def _():
                @pl.when(j % 2 == 0)
                def _():
                    box_impl(j, out0, osem0)

                @pl.when(j % 2 == 1)
                def _():
                    box_impl(j, out1, osem1)

            return carry

        lax.fori_loop(0, per_worker, box_body, 0)

        nv = N - base_box

        @pl.when(nv >= 1)
        def _():
            pltpu.make_async_copy(out_h.at[0], out0, osem0).wait()

        if per_worker >= 2:
            @pl.when(nv >= 2)
            def _():
                pltpu.make_async_copy(out_h.at[0], out1, osem1).wait()

    out = sc_pool(*flats, idx_t, wrow_t)
    return out.reshape(B, Nb, POOL, POOL, C)


# speedup vs baseline: 1.8441x; 1.1181x over previous
import functools

import jax
import jax.numpy as jnp
from jax import lax
from jax.experimental import pallas as pl
from jax.experimental.pallas import tpu as pltpu
from jax.experimental.pallas import tpu_sc as plsc

POOL = 7
PP = POOL * POOL
CH0 = 112
CH1 = 96
GPAD = CH0 + CH1
P0 = CH0 // 4
WROW = 128


def _prep(boxes, image_meta, feats):
    B, Nb, _ = boxes.shape
    C = feats[0].shape[-1]
    N = B * Nb

    hs = [f.shape[1] for f in feats]
    ws = [f.shape[2] for f in feats]

    fb = boxes.reshape(-1, 4)
    bw = fb[:, 3] - fb[:, 1]
    bh = fb[:, 2] - fb[:, 0]
    image_shape = image_meta[0, 4:7]
    image_area = image_shape[0] * image_shape[1]
    eq = jnp.log(jnp.sqrt(jnp.maximum(bw * bh, 1e-12)) * jnp.sqrt(image_area) / 224.0) / jnp.log(2.0)
    levels = jnp.maximum(2, jnp.minimum(4 + jnp.round(eq).astype(jnp.int32), 5))
    li = levels - 2

    h_i = jnp.take(jnp.asarray(hs, jnp.int32), li)
    w_i = jnp.take(jnp.asarray(ws, jnp.int32), li)
    hw_i = jnp.take(jnp.asarray([h * w for h, w in zip(hs, ws)], jnp.int32), li)
    bb = jnp.repeat(jnp.arange(B, dtype=jnp.int32), Nb)
    base = bb * hw_i

    hf = h_i.astype(jnp.float32)
    wf = w_i.astype(jnp.float32)
    g = jnp.arange(POOL, dtype=jnp.float32)
    ys = fb[:, 0][:, None] * (hf - 1.0)[:, None] + g[None, :] * ((fb[:, 2] - fb[:, 0]) * (hf - 1.0) / (POOL - 1))[:, None]
    xs = fb[:, 1][:, None] * (wf - 1.0)[:, None] + g[None, :] * ((fb[:, 3] - fb[:, 1]) * (wf - 1.0) / (POOL - 1))[:, None]
    y0f = jnp.floor(ys)
    x0f = jnp.floor(xs)
    y0 = jnp.clip(y0f.astype(jnp.int32), 0, h_i[:, None] - 1)
    y1 = jnp.clip(y0 + 1, 0, h_i[:, None] - 1)
    x0 = jnp.clip(x0f.astype(jnp.int32), 0, w_i[:, None] - 1)
    x1 = jnp.clip(x0 + 1, 0, w_i[:, None] - 1)
    wy = ys - y0f
    wx = xs - x0f

    ry0 = base[:, None] + y0 * w_i[:, None]
    ry1 = base[:, None] + y1 * w_i[:, None]
    tl = ry0[:, :, None] + x0[:, None, :]
    tr = ry0[:, :, None] + x1[:, None, :]
    bl = ry1[:, :, None] + x0[:, None, :]
    br = ry1[:, :, None] + x1[:, None, :]
    inter = jnp.stack([tl, tr, bl, br], axis=-1).reshape(N, 4 * PP).astype(jnp.int32)
    pad = jnp.broadcast_to(inter[:, -1:], (N, GPAD - 4 * PP))
    idx = jnp.concatenate([inter, pad], axis=1)

    wx_p = jnp.broadcast_to(wx[:, None, :], (N, POOL, POOL)).reshape(N, PP)
    wy_p = jnp.broadcast_to(wy[:, :, None], (N, POOL, POOL)).reshape(N, PP)
    zeros15 = jnp.zeros((N, 64 - PP), jnp.float32)
    zeros14 = jnp.zeros((N, 63 - PP), jnp.float32)
    lvl_f = li.astype(jnp.float32)[:, None]
    wrow = jnp.concatenate([wx_p, zeros14, lvl_f, wy_p, zeros15], axis=1)
    return idx, wrow


def kernel(boxes, image_meta, feat_p2, feat_p3, feat_p4, feat_p5):
    feats = [feat_p2, feat_p3, feat_p4, feat_p5]
    B, Nb, _ = boxes.shape
    C = feats[0].shape[-1]
    N = B * Nb
    assert C % 16 == 0

    idx, wrow = _prep(boxes, image_meta, feats)

    info = plsc.get_sparse_core_info()
    NC, NS = info.num_cores, info.num_subcores
    NW = NC * NS
    per_worker = -(-N // NW)
    NPAD = NW * per_worker
    npad = NPAD - N

    idx_t = jnp.concatenate([idx, jnp.zeros((npad, GPAD), jnp.int32)], axis=0)
    idx_t = idx_t.reshape(NW, per_worker * GPAD)
    wrow_t = jnp.concatenate([wrow, jnp.zeros((npad, WROW), jnp.float32)], axis=0)
    wrow_t = wrow_t.reshape(NW, per_worker * WROW)

    flats = [f.reshape(-1, C) for f in feats]

    mesh = plsc.VectorSubcoreMesh(core_axis_name="c", subcore_axis_name="s")

    @functools.partial(
        pl.kernel,
        mesh=mesh,
        out_type=jax.ShapeDtypeStruct((B, Nb, POOL, POOL, C), jnp.float32),
        scratch_types=[
            pltpu.VMEM((per_worker * GPAD,), jnp.int32),
            pltpu.VMEM((per_worker * WROW,), jnp.float32),
            pltpu.VMEM((CH0, C), jnp.float32),
            pltpu.VMEM((CH1, C), jnp.float32),
            pltpu.VMEM((POOL, POOL, C), jnp.float32),
            pltpu.VMEM((POOL, POOL, C), jnp.float32),
            pltpu.SemaphoreType.DMA,
            pltpu.SemaphoreType.DMA,
            pltpu.SemaphoreType.DMA,
            pltpu.SemaphoreType.DMA,
        ],
    )
    def sc_pool(fp2_h, fp3_h, fp4_h, fp5_h, idx_h, w_h, out_h,
                idx_v, w_v, buf0, buf1,
                out0, out1, sem0, sem1, osem0, osem1):
        wid = lax.axis_index("s") * NC + lax.axis_index("c")
        base_box = wid * per_worker
        fhs = [fp2_h, fp3_h, fp4_h, fp5_h]

        pltpu.sync_copy(idx_h.at[wid], idx_v)
        pltpu.sync_copy(w_h.at[wid], w_v)

        dnums = lax.GatherDimensionNumbers(
            offset_dims=(), collapsed_slice_dims=(0,), start_index_map=(0,))

        def bcast_lane(vec16, lane):
            idxv = jnp.broadcast_to(lane, (16,)).astype(jnp.int32)
            return lax.gather(vec16, idxv[:, None], dnums, (1,),
                              mode=lax.GatherScatterMode.PROMISE_IN_BOUNDS)

        def box_level(j):
            lv = w_v[pl.ds(j * WROW + 48, 16)]
            return lv[15]

        def gather0(j):
            lvl = box_level(j)
            for l, fh in enumerate(fhs):
                @pl.when(lvl == jnp.float32(l))
                def _():
                    pltpu.async_copy(
                        fh.at[idx_v.at[pl.ds(j * GPAD, CH0)]], buf0, sem0)

        def gather1(j):
            lvl = box_level(j)
            for l, fh in enumerate(fhs):
                @pl.when(lvl == jnp.float32(l))
                def _():
                    pltpu.async_copy(
                        fh.at[idx_v.at[pl.ds(j * GPAD + CH0, CH1)]],
                        buf1, sem1)

        @pl.when(base_box < N)
        def _():
            gather0(0)

        def combine(p, src, r, outb, wb):
            chunk = (p // 16) * 16
            lane = p - chunk
            wxc = w_v[pl.ds(wb + chunk, 16)]
            wyc = w_v[pl.ds(wb + 64 + chunk, 16)]
            wxp = bcast_lane(wxc, lane)
            wyp = bcast_lane(wyc, lane)
            for ch in range(C // 16):
                s = pl.ds(ch * 16, 16)
                tl = src[r, s]
                tr = src[r + 1, s]
                bl = src[r + 2, s]
                br = src[r + 3, s]
                top = tl + (tr - tl) * wxp
                bot = bl + (br - bl) * wxp
                outb[p // POOL, p % POOL, s] = top + (bot - top) * wyp

        def box_impl(j, outb, osem):
            wb = j * WROW

            gather1(j)
            pltpu.make_async_copy(fp2_h.at[pl.ds(0, CH0)], buf0, sem0).wait()

            def pt0(p, c):
                combine(p, buf0, p * 4, outb, wb)
                return c
            lax.fori_loop(0, P0, pt0, 0)

            pltpu.make_async_copy(fp2_h.at[pl.ds(0, CH1)], buf1, sem1).wait()

            @pl.when((j + 1 < per_worker) & (base_box + j + 1 < N))
            def _():
                gather0(j + 1)

            def pt1(p, c):
                combine(p, buf1, p * 4 - CH0, outb, wb)
                return c
            lax.fori_loop(P0, PP, pt1, 0)

            box = base_box + j

            @pl.when(j >= 2)
            def _():
                pltpu.make_async_copy(out_h.at[0, 0], outb, osem).wait()
            pltpu.async_copy(outb, out_h.at[box // Nb, box % Nb], osem)

        def box_body(j, carry):
            @pl.when(base_box + j < N)
            def _():
                @pl.when(j % 2 == 0)
                def _():
                    box_impl(j, out0, osem0)

                @pl.when(j % 2 == 1)
                def _():
                    box_impl(j, out1, osem1)

            return carry

        lax.fori_loop(0, per_worker, box_body, 0)

        nv = N - base_box

        @pl.when(nv >= 1)
        def _():
            pltpu.make_async_copy(out_h.at[0], out0, osem0).wait()

        if per_worker >= 2:
            @pl.when(nv >= 2)
            def _():
                pltpu.make_async_copy(out_h.at[0, 0], out1, osem1).wait()

    return sc_pool(*flats, idx_t, wrow_t)
